# baseline (speedup 1.0000x reference)
import jax
import jax.numpy as jnp
from jax.experimental import pallas as pl

KNN_K = 16


def _w2pers(point_xyz, camrotc2w, campos):
    xyz = jnp.matmul(point_xyz - campos[None, :], camrotc2w)
    x_pers = xyz[:, 0] / xyz[:, 2]
    y_pers = xyz[:, 1] / xyz[:, 2]
    z_pers = xyz[:, 2]
    return jnp.stack([x_pers, y_pers, z_pers], axis=-1)


def kernel(query_points, query_points_local, xyz_fov, points_embedding_fov, points_color_fov, points_dir_fov, camrotc2w, campos):
    q = query_points[0]
    NQ = q.shape[0]
    qq = q[:, 0] * q[:, 0] + q[:, 1] * q[:, 1] + q[:, 2] * q[:, 2]
    px, py, pz = xyz_fov[:, 0], xyz_fov[:, 1], xyz_fov[:, 2]
    pp = px * px + py * py + pz * pz
    qb = jax.lax.optimization_barrier(q.astype(jnp.bfloat16)).astype(jnp.float32)
    pxb = jax.lax.optimization_barrier(px.astype(jnp.bfloat16)).astype(jnp.float32)
    pyb = jax.lax.optimization_barrier(py.astype(jnp.bfloat16)).astype(jnp.float32)
    pzb = jax.lax.optimization_barrier(pz.astype(jnp.bfloat16)).astype(jnp.float32)
    dot = (qb[:, 0:1] * pxb[None, :] + qb[:, 1:2] * pyb[None, :]
           + qb[:, 2:3] * pzb[None, :])
    d_sq = (qq[:, None] + pp[None, :]) - 2.0 * dot
    neg_d, assign_index = jax.lax.top_k(-d_sq, KNN_K)
    ref_xyz = jnp.take(xyz_fov, assign_index, axis=0)
    ref_fea = jnp.take(points_embedding_fov, assign_index, axis=0)
    ref_col = jnp.take(points_color_fov, assign_index, axis=0)
    ref_dir = jnp.take(points_dir_fov, assign_index, axis=0)
    ref_xyz_pers = _w2pers(ref_xyz.reshape(-1, 3), camrotc2w, campos).reshape(1, NQ, KNN_K, 3)
    ref_xyz_b = ref_xyz[None, ...]
    xdist = ref_xyz_pers[..., 0] * ref_xyz_pers[..., 2] - query_points_local[:, :, None, 0] * query_points_local[:, :, None, 2]
    ydist = ref_xyz_pers[..., 1] * ref_xyz_pers[..., 2] - query_points_local[:, :, None, 1] * query_points_local[:, :, None, 2]
    zdist = ref_xyz_pers[..., 2] - query_points_local[:, :, None, 2]
    dists = jnp.stack([xdist, ydist, zdist], axis=-1)
    dists = jnp.concatenate([ref_xyz_b - query_points[:, :, None, :], dists], axis=-1)
    out = jnp.concatenate([dists, ref_fea[None, ...], ref_col[None, ...], ref_dir[None, ...]], axis=-1)
    return out


# speedup vs baseline: 1.7793x; 1.7793x over previous
import jax
import jax.numpy as jnp
from jax import lax
from jax.experimental import pallas as pl
from jax.experimental.pallas import tpu as pltpu
from jax.experimental.pallas import tpu_sc as plsc

KNN_K = 16
NQ = 4096
NP = 50000
EMBED = 64
OUTC = 6 + EMBED + 3 + 3

NC = 2
NS = 16
NW = NC * NS
QPW = NQ // NW
CHUNK = 2000
NCH = NP // CHUNK
VPC = CHUNK // 16
QG = 4
NGRP = QPW // QG
BQ = 32
NBC = QPW // BQ
BROWS = BQ * KNN_K

_INF = float("inf")

_GDN = lax.GatherDimensionNumbers(
    offset_dims=(), collapsed_slice_dims=(0,), start_index_map=(0,))


def _permute(v, idx):
    return lax.gather(v, idx[:, None], dimension_numbers=_GDN,
                      slice_sizes=(1,),
                      mode=lax.GatherScatterMode.PROMISE_IN_BOUNDS)


def _lane_min_scalar(v):
    ln = jnp.arange(16, dtype=jnp.int32)
    for s in (8, 4, 2, 1):
        v = jnp.minimum(v, _permute(v, ln ^ s))
    return v[0]


def _rnd_bf16(v):
    t = v * jnp.float32(65537.0)
    return t - (t - v)


def _merge(tk, ti, dv, cbase):
    ln = jnp.arange(16, dtype=jnp.int32)
    lm1 = jnp.maximum(ln - 1, 0)
    lane0 = ln == 0
    for l in range(16):
        dl = dv[l]
        dlb = jnp.full((16,), dl)
        ilb = jnp.full((16,), cbase + l, jnp.int32)
        tksh = jnp.where(lane0, -_INF, _permute(tk, lm1))
        tish = _permute(ti, lm1)
        c = tk <= dlb
        cs = tksh <= dlb
        tk = jnp.where(c, tk, jnp.where(cs, dlb, tksh))
        ti = jnp.where(c, ti, jnp.where(cs, ilb, tish))
    return tk, ti


def _sc_body(pxh, pyh, pzh, pph, qah, qsh, camh,
             at0, at1, at2, at3, at4, at5, at6, at7, at8, embh,
             fea_h, side_h,
             pxc, pyc, pzc, ppc, qtmpa, qtmpb, camtmp,
             tkd, tki, attv, embv, sidev, sem):
    cid = lax.axis_index("c")
    sid = lax.axis_index("s")
    wid = sid * NC + cid
    q0 = wid * QPW

    pltpu.sync_copy(qah.at[pl.ds(q0 * 4, QPW * 4)], qtmpa)
    pltpu.sync_copy(qsh.at[pl.ds(q0 * 16, QPW * 16)], qtmpb)
    pltpu.sync_copy(camh, camtmp)

    def _init(i, carry):
        tkd[pl.ds(i * 16, 16)] = jnp.full((16,), _INF, jnp.float32)
        tki[pl.ds(i * 16, 16)] = jnp.zeros((16,), jnp.int32)
        return carry
    lax.fori_loop(0, QPW, _init, 0)

    def _chunk(c, carry):
        base = c * CHUNK
        pltpu.sync_copy(pxh.at[pl.ds(base, CHUNK)], pxc)
        pltpu.sync_copy(pyh.at[pl.ds(base, CHUNK)], pyc)
        pltpu.sync_copy(pzh.at[pl.ds(base, CHUNK)], pzc)
        pltpu.sync_copy(pph.at[pl.ds(base, CHUNK)], ppc)

        def _group(g, gcarry):
            qi0 = g * QG
            qgv = qtmpa[pl.ds(g * 16, 16)]
            ths = []
            qs = []
            for k in range(QG):
                qi = qi0 + k
                ths.append(tkd[pl.ds(qi * 16, 16)][15])
                qs.append((qgv[4 * k], qgv[4 * k + 1], qgv[4 * k + 2],
                           qgv[4 * k + 3]))

            def _j(j, jcarry):
                o = j * 16
                pxv = pxc[pl.ds(o, 16)]
                pyv = pyc[pl.ds(o, 16)]
                pzv = pzc[pl.ds(o, 16)]
                ppv = ppc[pl.ds(o, 16)]
                cbase = base + o
                dvs = []
                dmin = None
                for k in range(QG):
                    qx, qy, qz, qqv = qs[k]
                    dot = (qx * pxv + qy * pyv) + qz * pzv
                    dvv = (qqv + ppv) - (dot + dot)
                    dvs.append(dvv)
                    delta = dvv - jcarry[k]
                    dmin = delta if dmin is None else jnp.minimum(dmin, delta)
                any_all = _lane_min_scalar(dmin) < 0.0

                def _do(_):
                    outs = []
                    for k in range(QG):
                        qi = qi0 + k
                        tk = tkd[pl.ds(qi * 16, 16)]
                        ti = tki[pl.ds(qi * 16, 16)]
                        tk2, ti2 = _merge(tk, ti, dvs[k], cbase)
                        tkd[pl.ds(qi * 16, 16)] = tk2
                        tki[pl.ds(qi * 16, 16)] = ti2
                        outs.append(tk2[15])
                    return tuple(outs)

                def _skip(_):
                    return tuple(jcarry)

                return lax.cond(any_all, _do, _skip, 0)

            lax.fori_loop(0, VPC, _j, tuple(ths))
            return gcarry
        lax.fori_loop(0, NGRP, _group, 0)
        return carry
    lax.fori_loop(0, NCH, _chunk, 0)

    camv = camtmp[pl.ds(0, 16)]
    cam_s = [camv[i] for i in range(12)]

    def _bc(b, carry):
        aths = (at0, at1, at2, at3, at4, at5, at6, at7, at8)
        cps = []
        for k in range(NBC):
            g = b * 4 + k
            idxr = tki.at[pl.ds(g * 128, 128)]
            cps.append(pltpu.async_copy(
                embh.at[idxr], embv.at[pl.ds(k * 128, 128)], sem))
            for a in range(9):
                cps.append(pltpu.async_copy(
                    aths[a].at[idxr],
                    attv.at[pl.ds(a * BROWS + k * 128, 128)], sem))
        for cp in cps:
            cp.wait()

        def _grp(t, gcarry):
            qi = b * BQ + t
            lr = t * 16
            ax = attv[pl.ds(lr, 16)]
            ay = attv[pl.ds(BROWS + lr, 16)]
            az = attv[pl.ds(2 * BROWS + lr, 16)]
            vq = qtmpb[pl.ds(qi * 16, 16)]
            qox = vq[0]
            qoy = vq[1]
            qoz = vq[2]
            qlxz = vq[3]
            qlyz = vq[4]
            qlz = vq[5]
            tx = _rnd_bf16(ax - cam_s[9])
            ty = _rnd_bf16(ay - cam_s[10])
            tz = _rnd_bf16(az - cam_s[11])
            xc = (tx * cam_s[0] + ty * cam_s[3]) + tz * cam_s[6]
            yc = (tx * cam_s[1] + ty * cam_s[4]) + tz * cam_s[7]
            zc = (tx * cam_s[2] + ty * cam_s[5]) + tz * cam_s[8]
            xp = xc / zc
            yp = yc / zc
            sidev[pl.ds(lr, 16)] = ax - qox
            sidev[pl.ds(BROWS + lr, 16)] = ay - qoy
            sidev[pl.ds(2 * BROWS + lr, 16)] = az - qoz
            sidev[pl.ds(3 * BROWS + lr, 16)] = xp * zc - qlxz
            sidev[pl.ds(4 * BROWS + lr, 16)] = yp * zc - qlyz
            sidev[pl.ds(5 * BROWS + lr, 16)] = zc - qlz
            for a in range(6):
                sidev[pl.ds((6 + a) * BROWS + lr, 16)] = (
                    attv[pl.ds((3 + a) * BROWS + lr, 16)])
            return gcarry
        lax.fori_loop(0, BQ, _grp, 0)

        gbase = wid * (QPW * KNN_K) + b * BROWS
        pltpu.sync_copy(embv, fea_h.at[pl.ds(gbase, BROWS)])
        for c in range(12):
            pltpu.sync_copy(
                sidev.at[pl.ds(c * BROWS, BROWS)],
                side_h.at[pl.ds(c * (NQ * KNN_K) + gbase, BROWS)])
        return carry
    lax.fori_loop(0, NBC, _bc, 0)


def kernel(query_points, query_points_local, xyz_fov, points_embedding_fov,
           points_color_fov, points_dir_fov, camrotc2w, campos):
    f32 = jnp.float32
    bar = jax.lax.optimization_barrier
    q = query_points[0]
    qpl = query_points_local[0]
    qq = jnp.sum(q * q, axis=-1)
    pp = jnp.sum(xyz_fov * xyz_fov, axis=-1)
    qb = bar(q.astype(jnp.bfloat16)).astype(f32)
    pb = bar(xyz_fov.astype(jnp.bfloat16)).astype(f32)
    rotb = bar(camrotc2w.astype(jnp.bfloat16)).astype(f32)
    qa = jnp.concatenate([qb, qq[:, None]], axis=-1).reshape(-1)
    qs = jnp.stack([q[:, 0], q[:, 1], q[:, 2],
                    qpl[:, 0] * qpl[:, 2], qpl[:, 1] * qpl[:, 2],
                    qpl[:, 2]], axis=-1)
    qs = jnp.pad(qs, ((0, 0), (0, 10))).reshape(-1)
    cam = jnp.concatenate([rotb.reshape(9), campos,
                           jnp.zeros((4,), f32)])
    atts = (xyz_fov[:, 0], xyz_fov[:, 1], xyz_fov[:, 2],
            points_color_fov[:, 0], points_color_fov[:, 1],
            points_color_fov[:, 2],
            points_dir_fov[:, 0], points_dir_fov[:, 1],
            points_dir_fov[:, 2])

    mesh = plsc.VectorSubcoreMesh(core_axis_name="c", subcore_axis_name="s")
    sc = pl.kernel(
        _sc_body,
        mesh=mesh,
        compiler_params=pltpu.CompilerParams(use_tc_tiling_on_sc=False),
        out_type=(
            jax.ShapeDtypeStruct((NQ * KNN_K, EMBED), f32),
            jax.ShapeDtypeStruct((12 * NQ * KNN_K,), f32),
        ),
        scratch_types=[
            pltpu.VMEM((CHUNK,), f32),
            pltpu.VMEM((CHUNK,), f32),
            pltpu.VMEM((CHUNK,), f32),
            pltpu.VMEM((CHUNK,), f32),
            pltpu.VMEM((QPW * 4,), f32),
            pltpu.VMEM((QPW * 16,), f32),
            pltpu.VMEM((16,), f32),
            pltpu.VMEM((QPW * 16,), f32),
            pltpu.VMEM((QPW * 16,), jnp.int32),
            pltpu.VMEM((9 * BROWS,), f32),
            pltpu.VMEM((BROWS, EMBED), f32),
            pltpu.VMEM((12 * BROWS,), f32),
            pltpu.SemaphoreType.DMA,
        ],
    )
    fea, side = sc(pb[:, 0], pb[:, 1], pb[:, 2], pp, qa, qs, cam, *atts,
                   points_embedding_fov)
    fea = fea.reshape(1, NQ, KNN_K, EMBED)
    side_t = side.reshape(12, NQ * KNN_K).T.reshape(1, NQ, KNN_K, 12)
    return jnp.concatenate([side_t[..., 0:6], fea, side_t[..., 6:12]],
                           axis=-1)


# speedup vs baseline: 2.0321x; 1.1421x over previous
import jax
import jax.numpy as jnp
from jax import lax
from jax.experimental import pallas as pl
from jax.experimental.pallas import tpu as pltpu
from jax.experimental.pallas import tpu_sc as plsc

KNN_K = 16
NQ = 4096
NP = 50000
EMBED = 64
OUTC = 6 + EMBED + 3 + 3

NC = 2
NS = 16
NW = NC * NS
QPW = NQ // NW
CHUNK = 2000
NCH = NP // CHUNK
VPC = CHUNK // 16
QG = 4
NGRP = QPW // QG
BQ = 32
NBC = QPW // BQ
BROWS = BQ * KNN_K

_INF = float("inf")

_GDN = lax.GatherDimensionNumbers(
    offset_dims=(), collapsed_slice_dims=(0,), start_index_map=(0,))


def _permute(v, idx):
    return lax.gather(v, idx[:, None], dimension_numbers=_GDN,
                      slice_sizes=(1,),
                      mode=lax.GatherScatterMode.PROMISE_IN_BOUNDS)


def _lane_min_scalar(v):
    ln = jnp.arange(16, dtype=jnp.int32)
    for s in (8, 4, 2, 1):
        v = jnp.minimum(v, _permute(v, ln ^ s))
    return v[0]


def _rnd_bf16(v):
    t = v * jnp.float32(65537.0)
    return t - (t - v)


def _merge(tk, ti, dv, cbase):
    ln = jnp.arange(16, dtype=jnp.int32)
    lm1 = jnp.maximum(ln - 1, 0)
    lane0 = ln == 0
    for l in range(16):
        dl = dv[l]
        dlb = jnp.full((16,), dl)
        ilb = jnp.full((16,), cbase + l, jnp.int32)
        tksh = jnp.where(lane0, -_INF, _permute(tk, lm1))
        tish = _permute(ti, lm1)
        c = tk <= dlb
        cs = tksh <= dlb
        tk = jnp.where(c, tk, jnp.where(cs, dlb, tksh))
        ti = jnp.where(c, ti, jnp.where(cs, ilb, tish))
    return tk, ti


def _sc_body(pxh, pyh, pzh, pph, qah, qsh, camh,
             at0, at1, at2, at3, at4, at5, at6, at7, at8, embh,
             fea_h, side_h,
             pxc, pyc, pzc, ppc, qtmpa, qtmpb, camtmp,
             tkd, tki, attv, embv, sidev, sem):
    cid = lax.axis_index("c")
    sid = lax.axis_index("s")
    wid = sid * NC + cid
    q0 = wid * QPW

    pltpu.sync_copy(qah.at[pl.ds(q0 * 4, QPW * 4)], qtmpa)
    pltpu.sync_copy(qsh.at[pl.ds(q0 * 16, QPW * 16)], qtmpb)
    pltpu.sync_copy(camh, camtmp)

    def _init(i, carry):
        tkd[pl.ds(i * 16, 16)] = jnp.full((16,), _INF, jnp.float32)
        tki[pl.ds(i * 16, 16)] = jnp.zeros((16,), jnp.int32)
        return carry
    lax.fori_loop(0, QPW, _init, 0)

    def _chunk(c, carry):
        base = c * CHUNK
        pltpu.sync_copy(pxh.at[pl.ds(base, CHUNK)], pxc)
        pltpu.sync_copy(pyh.at[pl.ds(base, CHUNK)], pyc)
        pltpu.sync_copy(pzh.at[pl.ds(base, CHUNK)], pzc)
        pltpu.sync_copy(pph.at[pl.ds(base, CHUNK)], ppc)

        def _group(g, gcarry):
            qi0 = g * QG
            qgv = qtmpa[pl.ds(g * 16, 16)]
            ths = []
            qs = []
            for k in range(QG):
                qi = qi0 + k
                ths.append(tkd[pl.ds(qi * 16, 16)][15])
                qs.append((qgv[4 * k], qgv[4 * k + 1], qgv[4 * k + 2],
                           qgv[4 * k + 3]))

            inf16 = jnp.full((16,), _INF, jnp.float32)

            def _check(ths_c, pdvs, pdmin, pbase):
                any_prev = _lane_min_scalar(pdmin) < 0.0

                def _do(_):
                    outs = []
                    for k in range(QG):
                        qi = qi0 + k
                        tk = tkd[pl.ds(qi * 16, 16)]
                        ti = tki[pl.ds(qi * 16, 16)]
                        tk2, ti2 = _merge(tk, ti, pdvs[k], pbase)
                        tkd[pl.ds(qi * 16, 16)] = tk2
                        tki[pl.ds(qi * 16, 16)] = ti2
                        outs.append(tk2[15])
                    return tuple(outs)

                def _skip(_):
                    return tuple(ths_c)

                return lax.cond(any_prev, _do, _skip, 0)

            def _j(j, jcarry):
                ths_c = jcarry[0]
                pdvs = jcarry[1]
                pdmin = jcarry[2]
                o = j * 16
                pxv = pxc[pl.ds(o, 16)]
                pyv = pyc[pl.ds(o, 16)]
                pzv = pzc[pl.ds(o, 16)]
                ppv = ppc[pl.ds(o, 16)]
                dvs = []
                dmin = None
                for k in range(QG):
                    qx, qy, qz, qqv = qs[k]
                    dot = (qx * pxv + qy * pyv) + qz * pzv
                    dvv = (qqv + ppv) - (dot + dot)
                    dvs.append(dvv)
                    delta = dvv - ths_c[k]
                    dmin = delta if dmin is None else jnp.minimum(dmin, delta)
                ths_n = _check(ths_c, pdvs, pdmin, base + (j - 1) * 16)
                return (ths_n, tuple(dvs), dmin)

            fin = lax.fori_loop(
                0, VPC, _j,
                (tuple(ths), (inf16, inf16, inf16, inf16), inf16))
            _check(fin[0], fin[1], fin[2], base + (VPC - 1) * 16)
            return gcarry
        lax.fori_loop(0, NGRP, _group, 0)
        return carry
    lax.fori_loop(0, NCH, _chunk, 0)

    camv = camtmp[pl.ds(0, 16)]
    cam_s = [camv[i] for i in range(12)]

    def _bc(b, carry):
        aths = (at0, at1, at2, at3, at4, at5, at6, at7, at8)
        cps = []
        for k in range(NBC):
            g = b * 4 + k
            idxr = tki.at[pl.ds(g * 128, 128)]
            cps.append(pltpu.async_copy(
                embh.at[idxr], embv.at[pl.ds(k * 128, 128)], sem))
            for a in range(9):
                cps.append(pltpu.async_copy(
                    aths[a].at[idxr],
                    attv.at[pl.ds(a * BROWS + k * 128, 128)], sem))
        for cp in cps:
            cp.wait()

        def _grp(t, gcarry):
            qi = b * BQ + t
            lr = t * 16
            ax = attv[pl.ds(lr, 16)]
            ay = attv[pl.ds(BROWS + lr, 16)]
            az = attv[pl.ds(2 * BROWS + lr, 16)]
            vq = qtmpb[pl.ds(qi * 16, 16)]
            qox = vq[0]
            qoy = vq[1]
            qoz = vq[2]
            qlxz = vq[3]
            qlyz = vq[4]
            qlz = vq[5]
            tx = _rnd_bf16(ax - cam_s[9])
            ty = _rnd_bf16(ay - cam_s[10])
            tz = _rnd_bf16(az - cam_s[11])
            xc = (tx * cam_s[0] + ty * cam_s[3]) + tz * cam_s[6]
            yc = (tx * cam_s[1] + ty * cam_s[4]) + tz * cam_s[7]
            zc = (tx * cam_s[2] + ty * cam_s[5]) + tz * cam_s[8]
            xp = xc / zc
            yp = yc / zc
            sidev[pl.ds(lr, 16)] = ax - qox
            sidev[pl.ds(BROWS + lr, 16)] = ay - qoy
            sidev[pl.ds(2 * BROWS + lr, 16)] = az - qoz
            sidev[pl.ds(3 * BROWS + lr, 16)] = xp * zc - qlxz
            sidev[pl.ds(4 * BROWS + lr, 16)] = yp * zc - qlyz
            sidev[pl.ds(5 * BROWS + lr, 16)] = zc - qlz
            for a in range(6):
                sidev[pl.ds((6 + a) * BROWS + lr, 16)] = (
                    attv[pl.ds((3 + a) * BROWS + lr, 16)])
            return gcarry
        lax.fori_loop(0, BQ, _grp, 0)

        gbase = wid * (QPW * KNN_K) + b * BROWS
        pltpu.sync_copy(embv, fea_h.at[pl.ds(gbase, BROWS)])
        for c in range(12):
            pltpu.sync_copy(
                sidev.at[pl.ds(c * BROWS, BROWS)],
                side_h.at[pl.ds(c * (NQ * KNN_K) + gbase, BROWS)])
        return carry
    lax.fori_loop(0, NBC, _bc, 0)


def kernel(query_points, query_points_local, xyz_fov, points_embedding_fov,
           points_color_fov, points_dir_fov, camrotc2w, campos):
    f32 = jnp.float32
    bar = jax.lax.optimization_barrier
    q = query_points[0]
    qpl = query_points_local[0]
    qq = jnp.sum(q * q, axis=-1)
    pp = jnp.sum(xyz_fov * xyz_fov, axis=-1)
    qb = bar(q.astype(jnp.bfloat16)).astype(f32)
    pb = bar(xyz_fov.astype(jnp.bfloat16)).astype(f32)
    rotb = bar(camrotc2w.astype(jnp.bfloat16)).astype(f32)
    qa = jnp.concatenate([qb, qq[:, None]], axis=-1).reshape(-1)
    qs = jnp.stack([q[:, 0], q[:, 1], q[:, 2],
                    qpl[:, 0] * qpl[:, 2], qpl[:, 1] * qpl[:, 2],
                    qpl[:, 2]], axis=-1)
    qs = jnp.pad(qs, ((0, 0), (0, 10))).reshape(-1)
    cam = jnp.concatenate([rotb.reshape(9), campos,
                           jnp.zeros((4,), f32)])
    atts = (xyz_fov[:, 0], xyz_fov[:, 1], xyz_fov[:, 2],
            points_color_fov[:, 0], points_color_fov[:, 1],
            points_color_fov[:, 2],
            points_dir_fov[:, 0], points_dir_fov[:, 1],
            points_dir_fov[:, 2])

    mesh = plsc.VectorSubcoreMesh(core_axis_name="c", subcore_axis_name="s")
    sc = pl.kernel(
        _sc_body,
        mesh=mesh,
        compiler_params=pltpu.CompilerParams(use_tc_tiling_on_sc=False),
        out_type=(
            jax.ShapeDtypeStruct((NQ * KNN_K, EMBED), f32),
            jax.ShapeDtypeStruct((12 * NQ * KNN_K,), f32),
        ),
        scratch_types=[
            pltpu.VMEM((CHUNK,), f32),
            pltpu.VMEM((CHUNK,), f32),
            pltpu.VMEM((CHUNK,), f32),
            pltpu.VMEM((CHUNK,), f32),
            pltpu.VMEM((QPW * 4,), f32),
            pltpu.VMEM((QPW * 16,), f32),
            pltpu.VMEM((16,), f32),
            pltpu.VMEM((QPW * 16,), f32),
            pltpu.VMEM((QPW * 16,), jnp.int32),
            pltpu.VMEM((9 * BROWS,), f32),
            pltpu.VMEM((BROWS, EMBED), f32),
            pltpu.VMEM((12 * BROWS,), f32),
            pltpu.SemaphoreType.DMA,
        ],
    )
    fea, side = sc(pb[:, 0], pb[:, 1], pb[:, 2], pp, qa, qs, cam, *atts,
                   points_embedding_fov)
    fea = fea.reshape(1, NQ, KNN_K, EMBED)
    side_t = side.reshape(12, NQ * KNN_K).T.reshape(1, NQ, KNN_K, 12)
    return jnp.concatenate([side_t[..., 0:6], fea, side_t[..., 6:12]],
                           axis=-1)


# speedup vs baseline: 2.3323x; 1.1477x over previous
import jax
import jax.numpy as jnp
from jax import lax
from jax.experimental import pallas as pl
from jax.experimental.pallas import tpu as pltpu
from jax.experimental.pallas import tpu_sc as plsc

KNN_K = 16
NQ = 4096
NP = 50000
EMBED = 64
OUTC = 6 + EMBED + 3 + 3

NC = 2
NS = 16
NW = NC * NS
QPW = NQ // NW
CHUNK = 2000
NCH = NP // CHUNK
VPC = CHUNK // 16
QG = 4
NGRP = QPW // QG
BQ = 32
NBC = QPW // BQ
BROWS = BQ * KNN_K

_INF = float("inf")

_GDN = lax.GatherDimensionNumbers(
    offset_dims=(), collapsed_slice_dims=(0,), start_index_map=(0,))


def _permute(v, idx):
    return lax.gather(v, idx[:, None], dimension_numbers=_GDN,
                      slice_sizes=(1,),
                      mode=lax.GatherScatterMode.PROMISE_IN_BOUNDS)


def _lane_min_scalar(v):
    ln = jnp.arange(16, dtype=jnp.int32)
    for s in (8, 4, 2, 1):
        v = jnp.minimum(v, _permute(v, ln ^ s))
    return v[0]


def _rnd_bf16(v):
    t = v * jnp.float32(65537.0)
    return t - (t - v)


def _merge(tk, ti, dv, cbase):
    ln = jnp.arange(16, dtype=jnp.int32)
    lm1 = jnp.maximum(ln - 1, 0)
    lane0 = ln == 0
    for l in range(16):
        dl = dv[l]
        dlb = jnp.full((16,), dl)
        ilb = jnp.full((16,), cbase + l, jnp.int32)
        tksh = jnp.where(lane0, -_INF, _permute(tk, lm1))
        tish = _permute(ti, lm1)
        c = tk <= dlb
        cs = tksh <= dlb
        tk = jnp.where(c, tk, jnp.where(cs, dlb, tksh))
        ti = jnp.where(c, ti, jnp.where(cs, ilb, tish))
    return tk, ti


def _sc_body(pxh, pyh, pzh, pph, qah, qsh, camh,
             at0, at1, at2, at3, at4, at5, at6, at7, at8, embh,
             fea_h, side_h,
             pxc, pyc, pzc, ppc, qtmpa, qtmpb, camtmp,
             tkd, tki, attv, embv, sidev, sem):
    cid = lax.axis_index("c")
    sid = lax.axis_index("s")
    wid = sid * NC + cid
    q0 = wid * QPW

    pltpu.sync_copy(qah.at[pl.ds(q0 * 4, QPW * 4)], qtmpa)
    pltpu.sync_copy(qsh.at[pl.ds(q0 * 16, QPW * 16)], qtmpb)
    pltpu.sync_copy(camh, camtmp)

    def _init(i, carry):
        tkd[pl.ds(i * 16, 16)] = jnp.full((16,), _INF, jnp.float32)
        tki[pl.ds(i * 16, 16)] = jnp.zeros((16,), jnp.int32)
        return carry
    lax.fori_loop(0, QPW, _init, 0)

    def _chunk(c, carry):
        base = c * CHUNK
        pltpu.sync_copy(pxh.at[pl.ds(base, CHUNK)], pxc)
        pltpu.sync_copy(pyh.at[pl.ds(base, CHUNK)], pyc)
        pltpu.sync_copy(pzh.at[pl.ds(base, CHUNK)], pzc)
        pltpu.sync_copy(pph.at[pl.ds(base, CHUNK)], ppc)

        def _group(g, gcarry):
            qi0 = g * QG
            qgv = qtmpa[pl.ds(g * 16, 16)]
            ths = []
            qs = []
            for k in range(QG):
                qi = qi0 + k
                ths.append(tkd[pl.ds(qi * 16, 16)][15])
                qs.append((qgv[4 * k], qgv[4 * k + 1], qgv[4 * k + 2],
                           qgv[4 * k + 3]))

            inf16 = jnp.full((16,), _INF, jnp.float32)

            def _scan_vreg(o, ths_c):
                pxv = pxc[pl.ds(o, 16)]
                pyv = pyc[pl.ds(o, 16)]
                pzv = pzc[pl.ds(o, 16)]
                ppv = ppc[pl.ds(o, 16)]
                dvs = []
                dmin = None
                for k in range(QG):
                    qx, qy, qz, qqv = qs[k]
                    dot = (qx * pxv + qy * pyv) + qz * pzv
                    dvv = (qqv + ppv) - (dot + dot)
                    dvs.append(dvv)
                    delta = dvv - ths_c[k]
                    dmin = delta if dmin is None else jnp.minimum(dmin, delta)
                return dvs, dmin

            def _merge_vreg(ths_c, dvs, dmin, cb):
                def _do(_):
                    outs = []
                    for k in range(QG):
                        qi = qi0 + k
                        tk = tkd[pl.ds(qi * 16, 16)]
                        ti = tki[pl.ds(qi * 16, 16)]
                        tk2, ti2 = _merge(tk, ti, dvs[k], cb)
                        tkd[pl.ds(qi * 16, 16)] = tk2
                        tki[pl.ds(qi * 16, 16)] = ti2
                        outs.append(tk2[15])
                    return tuple(outs)

                def _skip(_):
                    return tuple(ths_c)

                return lax.cond(_lane_min_scalar(dmin) < 0.0, _do, _skip, 0)

            def _mac(mj, ths_c):
                o0 = mj * 64
                acc = None
                for u in range(4):
                    _, dmin_u = _scan_vreg(o0 + u * 16, ths_c)
                    acc = dmin_u if acc is None else jnp.minimum(acc, dmin_u)

                def _do(_):
                    def _w(w, outs):
                        ow = o0 + w * 16
                        dvs_w, dmin_w = _scan_vreg(ow, outs)
                        return _merge_vreg(outs, dvs_w, dmin_w, base + ow)
                    return lax.fori_loop(0, 4, _w, tuple(ths_c))

                def _nochk(_):
                    return tuple(ths_c)

                return lax.cond(_lane_min_scalar(acc) < 0.0, _do, _nochk, 0)

            fin_ths = lax.fori_loop(0, (VPC - 1) // 4, _mac, tuple(ths))
            o_l = (VPC - 1) * 16
            dvs_l, dmin_l = _scan_vreg(o_l, fin_ths)
            _merge_vreg(fin_ths, dvs_l, dmin_l, base + o_l)
            return gcarry
        lax.fori_loop(0, NGRP, _group, 0)
        return carry
    lax.fori_loop(0, NCH, _chunk, 0)

    camv = camtmp[pl.ds(0, 16)]
    cam_s = [camv[i] for i in range(12)]

    def _bc(b, carry):
        aths = (at0, at1, at2, at3, at4, at5, at6, at7, at8)
        cps = []
        for k in range(NBC):
            g = b * 4 + k
            idxr = tki.at[pl.ds(g * 128, 128)]
            cps.append(pltpu.async_copy(
                embh.at[idxr], embv.at[pl.ds(k * 128, 128)], sem))
            for a in range(9):
                cps.append(pltpu.async_copy(
                    aths[a].at[idxr],
                    attv.at[pl.ds(a * BROWS + k * 128, 128)], sem))
        for cp in cps:
            cp.wait()

        def _grp(t, gcarry):
            qi = b * BQ + t
            lr = t * 16
            ax = attv[pl.ds(lr, 16)]
            ay = attv[pl.ds(BROWS + lr, 16)]
            az = attv[pl.ds(2 * BROWS + lr, 16)]
            vq = qtmpb[pl.ds(qi * 16, 16)]
            qox = vq[0]
            qoy = vq[1]
            qoz = vq[2]
            qlxz = vq[3]
            qlyz = vq[4]
            qlz = vq[5]
            tx = _rnd_bf16(ax - cam_s[9])
            ty = _rnd_bf16(ay - cam_s[10])
            tz = _rnd_bf16(az - cam_s[11])
            xc = (tx * cam_s[0] + ty * cam_s[3]) + tz * cam_s[6]
            yc = (tx * cam_s[1] + ty * cam_s[4]) + tz * cam_s[7]
            zc = (tx * cam_s[2] + ty * cam_s[5]) + tz * cam_s[8]
            xp = xc / zc
            yp = yc / zc
            sidev[pl.ds(lr, 16)] = ax - qox
            sidev[pl.ds(BROWS + lr, 16)] = ay - qoy
            sidev[pl.ds(2 * BROWS + lr, 16)] = az - qoz
            sidev[pl.ds(3 * BROWS + lr, 16)] = xp * zc - qlxz
            sidev[pl.ds(4 * BROWS + lr, 16)] = yp * zc - qlyz
            sidev[pl.ds(5 * BROWS + lr, 16)] = zc - qlz
            for a in range(6):
                sidev[pl.ds((6 + a) * BROWS + lr, 16)] = (
                    attv[pl.ds((3 + a) * BROWS + lr, 16)])
            return gcarry
        lax.fori_loop(0, BQ, _grp, 0)

        gbase = wid * (QPW * KNN_K) + b * BROWS
        pltpu.sync_copy(embv, fea_h.at[pl.ds(gbase, BROWS)])
        for c in range(12):
            pltpu.sync_copy(
                sidev.at[pl.ds(c * BROWS, BROWS)],
                side_h.at[pl.ds(c * (NQ * KNN_K) + gbase, BROWS)])
        return carry
    lax.fori_loop(0, NBC, _bc, 0)


def kernel(query_points, query_points_local, xyz_fov, points_embedding_fov,
           points_color_fov, points_dir_fov, camrotc2w, campos):
    f32 = jnp.float32
    bar = jax.lax.optimization_barrier
    q = query_points[0]
    qpl = query_points_local[0]
    qq = jnp.sum(q * q, axis=-1)
    pp = jnp.sum(xyz_fov * xyz_fov, axis=-1)
    qb = bar(q.astype(jnp.bfloat16)).astype(f32)
    pb = bar(xyz_fov.astype(jnp.bfloat16)).astype(f32)
    rotb = bar(camrotc2w.astype(jnp.bfloat16)).astype(f32)
    qa = jnp.concatenate([qb, qq[:, None]], axis=-1).reshape(-1)
    qs = jnp.stack([q[:, 0], q[:, 1], q[:, 2],
                    qpl[:, 0] * qpl[:, 2], qpl[:, 1] * qpl[:, 2],
                    qpl[:, 2]], axis=-1)
    qs = jnp.pad(qs, ((0, 0), (0, 10))).reshape(-1)
    cam = jnp.concatenate([rotb.reshape(9), campos,
                           jnp.zeros((4,), f32)])
    atts = (xyz_fov[:, 0], xyz_fov[:, 1], xyz_fov[:, 2],
            points_color_fov[:, 0], points_color_fov[:, 1],
            points_color_fov[:, 2],
            points_dir_fov[:, 0], points_dir_fov[:, 1],
            points_dir_fov[:, 2])

    mesh = plsc.VectorSubcoreMesh(core_axis_name="c", subcore_axis_name="s")
    sc = pl.kernel(
        _sc_body,
        mesh=mesh,
        compiler_params=pltpu.CompilerParams(use_tc_tiling_on_sc=False),
        out_type=(
            jax.ShapeDtypeStruct((NQ * KNN_K, EMBED), f32),
            jax.ShapeDtypeStruct((12 * NQ * KNN_K,), f32),
        ),
        scratch_types=[
            pltpu.VMEM((CHUNK,), f32),
            pltpu.VMEM((CHUNK,), f32),
            pltpu.VMEM((CHUNK,), f32),
            pltpu.VMEM((CHUNK,), f32),
            pltpu.VMEM((QPW * 4,), f32),
            pltpu.VMEM((QPW * 16,), f32),
            pltpu.VMEM((16,), f32),
            pltpu.VMEM((QPW * 16,), f32),
            pltpu.VMEM((QPW * 16,), jnp.int32),
            pltpu.VMEM((9 * BROWS,), f32),
            pltpu.VMEM((BROWS, EMBED), f32),
            pltpu.VMEM((12 * BROWS,), f32),
            pltpu.SemaphoreType.DMA,
        ],
    )
    fea, side = sc(pb[:, 0], pb[:, 1], pb[:, 2], pp, qa, qs, cam, *atts,
                   points_embedding_fov)
    fea = fea.reshape(1, NQ, KNN_K, EMBED)
    side_t = side.reshape(12, NQ * KNN_K).T.reshape(1, NQ, KNN_K, 12)
    return jnp.concatenate([side_t[..., 0:6], fea, side_t[..., 6:12]],
                           axis=-1)


# speedup vs baseline: 2.3463x; 1.0060x over previous
import jax
import jax.numpy as jnp
from jax import lax
from jax.experimental import pallas as pl
from jax.experimental.pallas import tpu as pltpu
from jax.experimental.pallas import tpu_sc as plsc

KNN_K = 16
NQ = 4096
NP = 50000
EMBED = 64
OUTC = 6 + EMBED + 3 + 3

NC = 2
NS = 16
NW = NC * NS
QPW = NQ // NW
CHUNK = 2000
NCH = NP // CHUNK
VPC = CHUNK // 16
QG = 4
NGRP = QPW // QG
BQ = 32
NBC = QPW // BQ
BROWS = BQ * KNN_K

_INF = float("inf")

_GDN = lax.GatherDimensionNumbers(
    offset_dims=(), collapsed_slice_dims=(0,), start_index_map=(0,))


def _permute(v, idx):
    return lax.gather(v, idx[:, None], dimension_numbers=_GDN,
                      slice_sizes=(1,),
                      mode=lax.GatherScatterMode.PROMISE_IN_BOUNDS)


def _lane_min_scalar(v):
    ln = jnp.arange(16, dtype=jnp.int32)
    for s in (8, 4, 2, 1):
        v = jnp.minimum(v, _permute(v, ln ^ s))
    return v[0]


def _rnd_bf16(v):
    t = v * jnp.float32(65537.0)
    return t - (t - v)


def _merge(tk, ti, dv, cbase):
    ln = jnp.arange(16, dtype=jnp.int32)
    lm1 = jnp.maximum(ln - 1, 0)
    lane0 = ln == 0
    for l in range(16):
        dl = dv[l]
        dlb = jnp.full((16,), dl)
        ilb = jnp.full((16,), cbase + l, jnp.int32)
        tksh = jnp.where(lane0, -_INF, _permute(tk, lm1))
        tish = _permute(ti, lm1)
        c = tk <= dlb
        cs = tksh <= dlb
        tk = jnp.where(c, tk, jnp.where(cs, dlb, tksh))
        ti = jnp.where(c, ti, jnp.where(cs, ilb, tish))
    return tk, ti


def _sc_body(pxh, pyh, pzh, pph, qah, qsh, camh,
             at0, at1, at2, at3, at4, at5, at6, at7, at8, embh,
             fea_h, side_h,
             pxc, pyc, pzc, ppc, qtmpa, qtmpb, camtmp,
             tkd, tki, attv, embv, sidev, sem):
    cid = lax.axis_index("c")
    sid = lax.axis_index("s")
    wid = sid * NC + cid
    q0 = wid * QPW

    pltpu.sync_copy(qah.at[pl.ds(q0 * 4, QPW * 4)], qtmpa)
    pltpu.sync_copy(qsh.at[pl.ds(q0 * 16, QPW * 16)], qtmpb)
    pltpu.sync_copy(camh, camtmp)

    def _init(i, carry):
        tkd[pl.ds(i * 16, 16)] = jnp.full((16,), _INF, jnp.float32)
        tki[pl.ds(i * 16, 16)] = jnp.zeros((16,), jnp.int32)
        return carry
    lax.fori_loop(0, QPW, _init, 0)

    def _chunk(c, carry):
        base = c * CHUNK
        pltpu.sync_copy(pxh.at[pl.ds(base, CHUNK)], pxc)
        pltpu.sync_copy(pyh.at[pl.ds(base, CHUNK)], pyc)
        pltpu.sync_copy(pzh.at[pl.ds(base, CHUNK)], pzc)
        pltpu.sync_copy(pph.at[pl.ds(base, CHUNK)], ppc)

        def _group(g, gcarry):
            qi0 = g * QG
            qgv = qtmpa[pl.ds(g * 16, 16)]
            ths = []
            qs = []
            for k in range(QG):
                qi = qi0 + k
                ths.append(tkd[pl.ds(qi * 16, 16)][15])
                qs.append((qgv[4 * k], qgv[4 * k + 1], qgv[4 * k + 2],
                           qgv[4 * k + 3]))

            inf16 = jnp.full((16,), _INF, jnp.float32)

            def _scan_vreg(o, ths_c):
                pxv = pxc[pl.ds(o, 16)]
                pyv = pyc[pl.ds(o, 16)]
                pzv = pzc[pl.ds(o, 16)]
                ppv = ppc[pl.ds(o, 16)]
                dvs = []
                dmin = None
                for k in range(QG):
                    qx, qy, qz, qqv = qs[k]
                    dot = (qx * pxv + qy * pyv) + qz * pzv
                    dvv = (qqv + ppv) - (dot + dot)
                    dvs.append(dvv)
                    delta = dvv - ths_c[k]
                    dmin = delta if dmin is None else jnp.minimum(dmin, delta)
                return dvs, dmin

            def _merge_vreg(ths_c, dvs, dmin, cb):
                def _do(_):
                    outs = []
                    for k in range(QG):
                        qi = qi0 + k
                        tk = tkd[pl.ds(qi * 16, 16)]
                        ti = tki[pl.ds(qi * 16, 16)]
                        tk2, ti2 = _merge(tk, ti, dvs[k], cb)
                        tkd[pl.ds(qi * 16, 16)] = tk2
                        tki[pl.ds(qi * 16, 16)] = ti2
                        outs.append(tk2[15])
                    return tuple(outs)

                def _skip(_):
                    return tuple(ths_c)

                return lax.cond(_lane_min_scalar(dmin) < 0.0, _do, _skip, 0)

            MAC = 5

            def _mac(mj, ths_c):
                o0 = mj * (MAC * 16)
                acc = None
                for u in range(MAC):
                    _, dmin_u = _scan_vreg(o0 + u * 16, ths_c)
                    acc = dmin_u if acc is None else jnp.minimum(acc, dmin_u)

                def _do(_):
                    def _w(w, outs):
                        ow = o0 + w * 16
                        dvs_w, dmin_w = _scan_vreg(ow, outs)
                        return _merge_vreg(outs, dvs_w, dmin_w, base + ow)
                    return lax.fori_loop(0, MAC, _w, tuple(ths_c))

                def _nochk(_):
                    return tuple(ths_c)

                return lax.cond(_lane_min_scalar(acc) < 0.0, _do, _nochk, 0)

            lax.fori_loop(0, VPC // MAC, _mac, tuple(ths))
            return gcarry
        lax.fori_loop(0, NGRP, _group, 0)
        return carry
    lax.fori_loop(0, NCH, _chunk, 0)

    camv = camtmp[pl.ds(0, 16)]
    cam_s = [camv[i] for i in range(12)]

    def _bc(b, carry):
        aths = (at0, at1, at2, at3, at4, at5, at6, at7, at8)
        cps = []
        for k in range(NBC):
            g = b * 4 + k
            idxr = tki.at[pl.ds(g * 128, 128)]
            cps.append(pltpu.async_copy(
                embh.at[idxr], embv.at[pl.ds(k * 128, 128)], sem))
            for a in range(9):
                cps.append(pltpu.async_copy(
                    aths[a].at[idxr],
                    attv.at[pl.ds(a * BROWS + k * 128, 128)], sem))
        for cp in cps:
            cp.wait()

        def _grp(t, gcarry):
            qi = b * BQ + t
            lr = t * 16
            ax = attv[pl.ds(lr, 16)]
            ay = attv[pl.ds(BROWS + lr, 16)]
            az = attv[pl.ds(2 * BROWS + lr, 16)]
            vq = qtmpb[pl.ds(qi * 16, 16)]
            qox = vq[0]
            qoy = vq[1]
            qoz = vq[2]
            qlxz = vq[3]
            qlyz = vq[4]
            qlz = vq[5]
            tx = _rnd_bf16(ax - cam_s[9])
            ty = _rnd_bf16(ay - cam_s[10])
            tz = _rnd_bf16(az - cam_s[11])
            xc = (tx * cam_s[0] + ty * cam_s[3]) + tz * cam_s[6]
            yc = (tx * cam_s[1] + ty * cam_s[4]) + tz * cam_s[7]
            zc = (tx * cam_s[2] + ty * cam_s[5]) + tz * cam_s[8]
            xp = xc / zc
            yp = yc / zc
            sidev[pl.ds(lr, 16)] = ax - qox
            sidev[pl.ds(BROWS + lr, 16)] = ay - qoy
            sidev[pl.ds(2 * BROWS + lr, 16)] = az - qoz
            sidev[pl.ds(3 * BROWS + lr, 16)] = xp * zc - qlxz
            sidev[pl.ds(4 * BROWS + lr, 16)] = yp * zc - qlyz
            sidev[pl.ds(5 * BROWS + lr, 16)] = zc - qlz
            for a in range(6):
                sidev[pl.ds((6 + a) * BROWS + lr, 16)] = (
                    attv[pl.ds((3 + a) * BROWS + lr, 16)])
            return gcarry
        lax.fori_loop(0, BQ, _grp, 0)

        gbase = wid * (QPW * KNN_K) + b * BROWS
        pltpu.sync_copy(embv, fea_h.at[pl.ds(gbase, BROWS)])
        for c in range(12):
            pltpu.sync_copy(
                sidev.at[pl.ds(c * BROWS, BROWS)],
                side_h.at[pl.ds(c * (NQ * KNN_K) + gbase, BROWS)])
        return carry
    lax.fori_loop(0, NBC, _bc, 0)


def kernel(query_points, query_points_local, xyz_fov, points_embedding_fov,
           points_color_fov, points_dir_fov, camrotc2w, campos):
    f32 = jnp.float32
    bar = jax.lax.optimization_barrier
    q = query_points[0]
    qpl = query_points_local[0]
    qq = jnp.sum(q * q, axis=-1)
    pp = jnp.sum(xyz_fov * xyz_fov, axis=-1)
    qb = bar(q.astype(jnp.bfloat16)).astype(f32)
    pb = bar(xyz_fov.astype(jnp.bfloat16)).astype(f32)
    rotb = bar(camrotc2w.astype(jnp.bfloat16)).astype(f32)
    qa = jnp.concatenate([qb, qq[:, None]], axis=-1).reshape(-1)
    qs = jnp.stack([q[:, 0], q[:, 1], q[:, 2],
                    qpl[:, 0] * qpl[:, 2], qpl[:, 1] * qpl[:, 2],
                    qpl[:, 2]], axis=-1)
    qs = jnp.pad(qs, ((0, 0), (0, 10))).reshape(-1)
    cam = jnp.concatenate([rotb.reshape(9), campos,
                           jnp.zeros((4,), f32)])
    atts = (xyz_fov[:, 0], xyz_fov[:, 1], xyz_fov[:, 2],
            points_color_fov[:, 0], points_color_fov[:, 1],
            points_color_fov[:, 2],
            points_dir_fov[:, 0], points_dir_fov[:, 1],
            points_dir_fov[:, 2])

    mesh = plsc.VectorSubcoreMesh(core_axis_name="c", subcore_axis_name="s")
    sc = pl.kernel(
        _sc_body,
        mesh=mesh,
        compiler_params=pltpu.CompilerParams(use_tc_tiling_on_sc=False),
        out_type=(
            jax.ShapeDtypeStruct((NQ * KNN_K, EMBED), f32),
            jax.ShapeDtypeStruct((12 * NQ * KNN_K,), f32),
        ),
        scratch_types=[
            pltpu.VMEM((CHUNK,), f32),
            pltpu.VMEM((CHUNK,), f32),
            pltpu.VMEM((CHUNK,), f32),
            pltpu.VMEM((CHUNK,), f32),
            pltpu.VMEM((QPW * 4,), f32),
            pltpu.VMEM((QPW * 16,), f32),
            pltpu.VMEM((16,), f32),
            pltpu.VMEM((QPW * 16,), f32),
            pltpu.VMEM((QPW * 16,), jnp.int32),
            pltpu.VMEM((9 * BROWS,), f32),
            pltpu.VMEM((BROWS, EMBED), f32),
            pltpu.VMEM((12 * BROWS,), f32),
            pltpu.SemaphoreType.DMA,
        ],
    )
    fea, side = sc(pb[:, 0], pb[:, 1], pb[:, 2], pp, qa, qs, cam, *atts,
                   points_embedding_fov)
    fea = fea.reshape(1, NQ, KNN_K, EMBED)
    side_t = side.reshape(12, NQ * KNN_K).T.reshape(1, NQ, KNN_K, 12)
    return jnp.concatenate([side_t[..., 0:6], fea, side_t[..., 6:12]],
                           axis=-1)
